# baseline (device time: 12812 ns/iter reference)
import jax
import jax.numpy as jnp
from jax import lax
from jax.experimental import pallas as pl
from jax.experimental.pallas import tpu as pltpu

N_DEV = 4
B, SQ, HQ, DH = 2, 128, 4, 64
QBLK = 64
R = B * SQ


def kernel(x, Wq, K_ext, V_ext, Wo):
    x2 = x.reshape(R, 512)
    kmat = K_ext.reshape(R, HQ * DH).astype(jnp.bfloat16)
    vmat = V_ext.reshape(R, HQ * DH).astype(jnp.bfloat16)

    def body(x_ref, wq_ref, k_ref, v_ref, wo_hbm, out_ref,
             krem_ref, vrem_ref, ctx_ref, wo_ref, send_sems, recv_sems,
             wo_sem):
        my = lax.axis_index("i")
        partner = (my + 2) % N_DEV

        barrier_sem = pltpu.get_barrier_semaphore()
        pl.semaphore_signal(
            barrier_sem, inc=1,
            device_id=(partner,), device_id_type=pl.DeviceIdType.MESH,
        )
        pl.semaphore_wait(barrier_sem, 1)

        k_rdma = pltpu.make_async_remote_copy(
            src_ref=k_ref, dst_ref=krem_ref,
            send_sem=send_sems.at[0], recv_sem=recv_sems.at[0],
            device_id=(partner,), device_id_type=pl.DeviceIdType.MESH,
        )
        v_rdma = pltpu.make_async_remote_copy(
            src_ref=v_ref, dst_ref=vrem_ref,
            send_sem=send_sems.at[1], recv_sem=recv_sems.at[1],
            device_id=(partner,), device_id_type=pl.DeviceIdType.MESH,
        )
        k_rdma.start()
        v_rdma.start()

        wo_copy = pltpu.make_async_copy(wo_hbm, wo_ref, wo_sem)
        wo_copy.start()

        row_id = lax.broadcasted_iota(jnp.int32, (R, R), 0) // QBLK
        col_id = lax.broadcasted_iota(jnp.int32, (R, R), 1) // QBLK
        mask = (row_id == col_id).astype(jnp.float32)

        q_all = jnp.dot(
            x_ref[...].astype(jnp.bfloat16),
            wq_ref[...].astype(jnp.bfloat16),
            preferred_element_type=jnp.float32,
        )
        q_s = (q_all * 0.125).astype(jnp.bfloat16)

        def head(val, h):
            return val[:, h * DH:(h + 1) * DH]

        def half(qh, kfull_ref, vfull_ref, h):
            s = lax.dot_general(
                qh, kfull_ref[:, h * DH:(h + 1) * DH],
                (((1,), (1,)), ((), ())),
                preferred_element_type=jnp.float32,
            )
            e = jnp.exp(s) * mask
            d = jnp.sum(e, axis=-1, keepdims=True)
            c = jnp.dot(
                e.astype(jnp.bfloat16), vfull_ref[:, h * DH:(h + 1) * DH],
                preferred_element_type=jnp.float32,
            )
            return d, c

        loc = [half(head(q_s, h), k_ref, v_ref, h) for h in range(HQ)]

        k_rdma.wait_recv()
        v_rdma.wait_recv()
        for h in range(HQ):
            d_l, c_l = loc[h]
            d_r, c_r = half(head(q_s, h), krem_ref, vrem_ref, h)
            ctx_ref[:, h * DH:(h + 1) * DH] = (
                (c_l + c_r) * (1.0 / (d_l + d_r))
            ).astype(jnp.bfloat16)

        wo_copy.wait()
        out_ref[...] = jnp.dot(
            ctx_ref[...], wo_ref[...].astype(jnp.bfloat16),
            preferred_element_type=jnp.float32,
        ).astype(jnp.bfloat16)

        k_rdma.wait_send()
        v_rdma.wait_send()

    out = pl.pallas_call(
        body,
        out_shape=jax.ShapeDtypeStruct((R, 512), jnp.bfloat16),
        in_specs=[
            pl.BlockSpec(memory_space=pltpu.VMEM),
            pl.BlockSpec(memory_space=pltpu.VMEM),
            pl.BlockSpec(memory_space=pltpu.VMEM),
            pl.BlockSpec(memory_space=pltpu.VMEM),
            pl.BlockSpec(memory_space=pl.ANY),
        ],
        out_specs=pl.BlockSpec(memory_space=pltpu.VMEM),
        scratch_shapes=[
            pltpu.VMEM((R, HQ * DH), jnp.bfloat16),
            pltpu.VMEM((R, HQ * DH), jnp.bfloat16),
            pltpu.VMEM((R, HQ * DH), jnp.bfloat16),
            pltpu.VMEM((HQ * DH, 512), jnp.float32),
            pltpu.SemaphoreType.DMA((2,)),
            pltpu.SemaphoreType.DMA((2,)),
            pltpu.SemaphoreType.DMA,
        ],
        compiler_params=pltpu.CompilerParams(collective_id=0),
    )(x2, Wq, kmat, vmat, Wo)
    return out.reshape(B, SQ, 512)


# device time: 12426 ns/iter; 1.0311x vs baseline; 1.0311x over previous
import jax
import jax.numpy as jnp
from jax import lax
from jax.experimental import pallas as pl
from jax.experimental.pallas import tpu as pltpu

N_DEV = 4
B, SQ, HQ, DH = 2, 128, 4, 64
QBLK = 64
NQB = SQ // QBLK


def kernel(x, Wq, K_ext, V_ext, Wo):
    x2 = x.reshape(B * SQ, 512)
    kmat = K_ext.reshape(B * SQ, HQ * DH).astype(jnp.bfloat16)
    vmat = V_ext.reshape(B * SQ, HQ * DH).astype(jnp.bfloat16)

    def body(x_hbm, wq_hbm, k_ref, v_ref, wo_hbm, out_ref,
             krem_ref, vrem_ref, ctx_ref, wo_ref, x_ref, wq_ref,
             send_sems, recv_sems, wo_sem, x_sem, wq_sem):
        my = lax.axis_index("i")
        partner = (my + 2) % N_DEV

        x_copy = pltpu.make_async_copy(x_hbm, x_ref, x_sem)
        wq_copy = pltpu.make_async_copy(wq_hbm, wq_ref, wq_sem)
        wo_copy = pltpu.make_async_copy(wo_hbm, wo_ref, wo_sem)
        x_copy.start()
        wq_copy.start()
        wo_copy.start()

        barrier_sem = pltpu.get_barrier_semaphore()
        pl.semaphore_signal(
            barrier_sem, inc=1,
            device_id=(partner,), device_id_type=pl.DeviceIdType.MESH,
        )
        pl.semaphore_wait(barrier_sem, 1)

        k_rdma = pltpu.make_async_remote_copy(
            src_ref=k_ref, dst_ref=krem_ref,
            send_sem=send_sems.at[0], recv_sem=recv_sems.at[0],
            device_id=(partner,), device_id_type=pl.DeviceIdType.MESH,
        )
        v_rdma = pltpu.make_async_remote_copy(
            src_ref=v_ref, dst_ref=vrem_ref,
            send_sem=send_sems.at[1], recv_sem=recv_sems.at[1],
            device_id=(partner,), device_id_type=pl.DeviceIdType.MESH,
        )
        k_rdma.start()
        v_rdma.start()

        x_copy.wait()
        wq_copy.wait()
        q_all = jnp.dot(
            x_ref[...].astype(jnp.bfloat16),
            wq_ref[...].astype(jnp.bfloat16),
            preferred_element_type=jnp.float32,
        )
        q_s = (q_all * 0.125).astype(jnp.bfloat16)

        def blk(ref_or_val, b, j, h):
            r0 = b * SQ + j * QBLK
            return ref_or_val[r0:r0 + QBLK, h * DH:(h + 1) * DH]

        loc = {}
        for b in range(B):
            for h in range(HQ):
                for j in range(NQB):
                    s_l = lax.dot_general(
                        blk(q_s, b, j, h), blk(k_ref, b, j, h),
                        (((1,), (1,)), ((), ())),
                        preferred_element_type=jnp.float32,
                    )
                    e_l = jnp.exp(s_l)
                    d_l = jnp.sum(e_l, axis=-1, keepdims=True)
                    c_l = jnp.dot(
                        e_l.astype(jnp.bfloat16), blk(v_ref, b, j, h),
                        preferred_element_type=jnp.float32,
                    )
                    loc[b, h, j] = (d_l, c_l)

        k_rdma.wait_recv()
        rem = {}
        for b in range(B):
            for h in range(HQ):
                for j in range(NQB):
                    s_r = lax.dot_general(
                        blk(q_s, b, j, h), blk(krem_ref, b, j, h),
                        (((1,), (1,)), ((), ())),
                        preferred_element_type=jnp.float32,
                    )
                    e_r = jnp.exp(s_r)
                    d_r = jnp.sum(e_r, axis=-1, keepdims=True)
                    rem[b, h, j] = (e_r.astype(jnp.bfloat16), d_r)

        v_rdma.wait_recv()
        for b in range(B):
            for h in range(HQ):
                for j in range(NQB):
                    d_l, c_l = loc[b, h, j]
                    e_r, d_r = rem[b, h, j]
                    c_r = jnp.dot(
                        e_r, blk(vrem_ref, b, j, h),
                        preferred_element_type=jnp.float32,
                    )
                    c = (c_l + c_r) * (1.0 / (d_l + d_r))
                    r0 = b * SQ + j * QBLK
                    ctx_ref[r0:r0 + QBLK, h * DH:(h + 1) * DH] = (
                        c.astype(jnp.bfloat16)
                    )

        wo_copy.wait()
        out_ref[...] = jnp.dot(
            ctx_ref[...], wo_ref[...].astype(jnp.bfloat16),
            preferred_element_type=jnp.float32,
        ).astype(jnp.bfloat16)

        k_rdma.wait_send()
        v_rdma.wait_send()

    out = pl.pallas_call(
        body,
        out_shape=jax.ShapeDtypeStruct((B * SQ, 512), jnp.bfloat16),
        in_specs=[
            pl.BlockSpec(memory_space=pl.ANY),
            pl.BlockSpec(memory_space=pl.ANY),
            pl.BlockSpec(memory_space=pltpu.VMEM),
            pl.BlockSpec(memory_space=pltpu.VMEM),
            pl.BlockSpec(memory_space=pl.ANY),
        ],
        out_specs=pl.BlockSpec(memory_space=pltpu.VMEM),
        scratch_shapes=[
            pltpu.VMEM((B * SQ, HQ * DH), jnp.bfloat16),
            pltpu.VMEM((B * SQ, HQ * DH), jnp.bfloat16),
            pltpu.VMEM((B * SQ, HQ * DH), jnp.bfloat16),
            pltpu.VMEM((HQ * DH, 512), jnp.float32),
            pltpu.VMEM((B * SQ, 512), jnp.float32),
            pltpu.VMEM((512, HQ * DH), jnp.float32),
            pltpu.SemaphoreType.DMA((2,)),
            pltpu.SemaphoreType.DMA((2,)),
            pltpu.SemaphoreType.DMA,
            pltpu.SemaphoreType.DMA,
            pltpu.SemaphoreType.DMA,
        ],
        compiler_params=pltpu.CompilerParams(collective_id=0),
    )(x2, Wq, kmat, vmat, Wo)
    return out.reshape(B, SQ, 512)


# device time: 11518 ns/iter; 1.1123x vs baseline; 1.0788x over previous
import jax
import jax.numpy as jnp
from jax import lax
from jax.experimental import pallas as pl
from jax.experimental.pallas import tpu as pltpu

N_DEV = 4
B, SQ, HQ, DH = 2, 128, 4, 64
QBLK = 64
NQB = SQ // QBLK


def kernel(x, Wq, K_ext, V_ext, Wo):
    kv = jnp.concatenate([K_ext, V_ext], axis=0).reshape(
        2 * B * SQ, HQ * DH).astype(jnp.bfloat16)

    def body(x_hbm, wq_hbm, kv_ref, wo_hbm, out_ref,
             krem_ref, vrem_ref, ctx_ref, wo_ref, x_ref, wq_ref,
             send_sems, recv_sems, wo_sem, x_sem, wq_sem):
        k_ref = kv_ref.at[0:B * SQ, :]
        v_ref = kv_ref.at[B * SQ:2 * B * SQ, :]
        my = lax.axis_index("i")
        partner = (my + 2) % N_DEV

        x_copy = pltpu.make_async_copy(x_hbm, x_ref, x_sem)
        wq_copy = pltpu.make_async_copy(wq_hbm, wq_ref, wq_sem)
        wo_copy = pltpu.make_async_copy(wo_hbm, wo_ref, wo_sem)
        x_copy.start()
        wq_copy.start()
        wo_copy.start()

        barrier_sem = pltpu.get_barrier_semaphore()
        pl.semaphore_signal(
            barrier_sem, inc=1,
            device_id=(partner,), device_id_type=pl.DeviceIdType.MESH,
        )
        pl.semaphore_wait(barrier_sem, 1)

        k_rdma = pltpu.make_async_remote_copy(
            src_ref=k_ref, dst_ref=krem_ref,
            send_sem=send_sems.at[0], recv_sem=recv_sems.at[0],
            device_id=(partner,), device_id_type=pl.DeviceIdType.MESH,
        )
        v_rdma = pltpu.make_async_remote_copy(
            src_ref=v_ref, dst_ref=vrem_ref,
            send_sem=send_sems.at[1], recv_sem=recv_sems.at[1],
            device_id=(partner,), device_id_type=pl.DeviceIdType.MESH,
        )
        k_rdma.start()
        v_rdma.start()

        x_copy.wait()
        wq_copy.wait()
        q_all = jnp.dot(
            x_ref[...].reshape(B * SQ, 512).astype(jnp.bfloat16),
            wq_ref[...].astype(jnp.bfloat16),
            preferred_element_type=jnp.float32,
        )
        q_s = (q_all * 0.125).astype(jnp.bfloat16)

        def blk(ref_or_val, b, j, h):
            r0 = b * SQ + j * QBLK
            return ref_or_val[r0:r0 + QBLK, h * DH:(h + 1) * DH]

        loc = {}
        for b in range(B):
            for h in range(HQ):
                for j in range(NQB):
                    s_l = lax.dot_general(
                        blk(q_s, b, j, h), blk(k_ref, b, j, h),
                        (((1,), (1,)), ((), ())),
                        preferred_element_type=jnp.float32,
                    )
                    e_l = jnp.exp(s_l)
                    d_l = jnp.sum(e_l, axis=-1, keepdims=True)
                    c_l = jnp.dot(
                        e_l.astype(jnp.bfloat16), blk(v_ref, b, j, h),
                        preferred_element_type=jnp.float32,
                    )
                    loc[b, h, j] = (d_l, c_l)

        k_rdma.wait_recv()
        rem = {}
        for b in range(B):
            for h in range(HQ):
                for j in range(NQB):
                    s_r = lax.dot_general(
                        blk(q_s, b, j, h), blk(krem_ref, b, j, h),
                        (((1,), (1,)), ((), ())),
                        preferred_element_type=jnp.float32,
                    )
                    e_r = jnp.exp(s_r)
                    d_r = jnp.sum(e_r, axis=-1, keepdims=True)
                    rem[b, h, j] = (e_r.astype(jnp.bfloat16), d_r)

        v_rdma.wait_recv()
        for b in range(B):
            for h in range(HQ):
                for j in range(NQB):
                    d_l, c_l = loc[b, h, j]
                    e_r, d_r = rem[b, h, j]
                    c_r = jnp.dot(
                        e_r, blk(vrem_ref, b, j, h),
                        preferred_element_type=jnp.float32,
                    )
                    c = (c_l + c_r) * (1.0 / (d_l + d_r))
                    r0 = b * SQ + j * QBLK
                    ctx_ref[r0:r0 + QBLK, h * DH:(h + 1) * DH] = (
                        c.astype(jnp.bfloat16)
                    )

        wo_copy.wait()
        out_ref[...] = jnp.dot(
            ctx_ref[...], wo_ref[...].astype(jnp.bfloat16),
            preferred_element_type=jnp.float32,
        ).astype(jnp.bfloat16)

        k_rdma.wait_send()
        v_rdma.wait_send()

    out = pl.pallas_call(
        body,
        out_shape=jax.ShapeDtypeStruct((B * SQ, 512), jnp.bfloat16),
        in_specs=[
            pl.BlockSpec(memory_space=pl.ANY),
            pl.BlockSpec(memory_space=pl.ANY),
            pl.BlockSpec(memory_space=pltpu.VMEM),
            pl.BlockSpec(memory_space=pl.ANY),
        ],
        out_specs=pl.BlockSpec(memory_space=pltpu.VMEM),
        scratch_shapes=[
            pltpu.VMEM((B * SQ, HQ * DH), jnp.bfloat16),
            pltpu.VMEM((B * SQ, HQ * DH), jnp.bfloat16),
            pltpu.VMEM((B * SQ, HQ * DH), jnp.bfloat16),
            pltpu.VMEM((HQ * DH, 512), jnp.float32),
            pltpu.VMEM((B, SQ, 512), jnp.float32),
            pltpu.VMEM((512, HQ * DH), jnp.float32),
            pltpu.SemaphoreType.DMA((2,)),
            pltpu.SemaphoreType.DMA((2,)),
            pltpu.SemaphoreType.DMA,
            pltpu.SemaphoreType.DMA,
            pltpu.SemaphoreType.DMA,
        ],
        compiler_params=pltpu.CompilerParams(collective_id=0),
    )(x, Wq, kv, Wo)
    return out.reshape(B, SQ, 512)
